# Initial kernel scaffold; baseline (speedup 1.0000x reference)
import jax
import jax.numpy as jnp
from jax.experimental import pallas as pl


def kernel(tens):
    raise NotImplementedError("write your pallas kernel here")


# speedup vs baseline: 20.6919x; 20.6919x over previous
import functools

import jax
import jax.numpy as jnp
from jax import lax
from jax.experimental import pallas as pl
from jax.experimental.pallas import tpu as pltpu
from jax.experimental.pallas import tpu_sc as plsc

R = 128
N = 32768
K = 64
L = 16
CH = 128
NCH = N // CH
NSUP = 16
BIG = 1 << 30
NEG = float("-inf")


def _row_topk(row_v, cmax_v, vals_s, poss_s, outb_v):
    iota = lax.iota(jnp.int32, L)
    neg_vec = jnp.full((L,), NEG, jnp.float32)

    def sup_body(s, U):
        def ch_body(j, accv):
            base = (s * L + j) * CH
            m = row_v[pl.ds(base, L)]
            for k in range(1, CH // L):
                m = jnp.maximum(m, row_v[pl.ds(base + k * L, L)])
            return jnp.where(iota == j, jnp.max(m), accv)

        accv = lax.fori_loop(0, L, ch_body, neg_vec)
        cmax_v[pl.ds(s * L, L)] = accv
        return jnp.where(iota == s, jnp.max(accv), U)

    U = lax.fori_loop(0, NSUP, sup_body, neg_vec)

    def ext_body(i, U):
        m = jnp.max(U)
        s = jnp.min(jnp.where(U == m, iota, BIG))
        t = cmax_v[pl.ds(s * L, L)]
        c16 = jnp.min(jnp.where(t == m, iota, BIG))
        base = (s * L + c16) * CH
        vs = [row_v[pl.ds(base + k * L, L)] for k in range(CH // L)]
        pos = BIG
        for k in range(CH // L):
            pos = jnp.minimum(
                pos, jnp.min(jnp.where(vs[k] == m, iota + (base + k * L), BIG)))
        nm = neg_vec
        for k in range(CH // L):
            w = jnp.where(iota + (base + k * L) == pos, NEG, vs[k])
            row_v[pl.ds(base + k * L, L)] = w
            nm = jnp.maximum(nm, w)
        t2 = jnp.where(iota == c16, jnp.max(nm), t)
        cmax_v[pl.ds(s * L, L)] = t2
        vals_s[i] = m
        poss_s[i] = pos
        return jnp.where(iota == s, jnp.max(t2), U)

    lax.fori_loop(0, K, ext_body, U)

    maxp = poss_s[0]

    def rolled(r):
        return jnp.bitwise_and(poss_s[r] - maxp, N - 1)

    Rv, Vv = [], []
    for a in range(K // L):
        def ins_body(li, carry):
            Ra, Va = carry
            r = a * L + li
            Ra = jnp.where(iota == li, rolled(r), Ra)
            Va = jnp.where(iota == li, vals_s[r], Va)
            return Ra, Va

        Ra, Va = lax.fori_loop(
            0, L, ins_body,
            (jnp.zeros((L,), jnp.int32), jnp.zeros((L,), jnp.float32)))
        Rv.append(Ra)
        Vv.append(Va)

    def rank_body(r, Ks):
        sr = rolled(r)
        return tuple(
            Ka + jnp.where(Ra > sr, 1, 0).astype(jnp.int32)
            for Ka, Ra in zip(Ks, Rv))

    Ks = lax.fori_loop(0, K, rank_body,
                       tuple(jnp.zeros((L,), jnp.int32) for _ in range(K // L)))
    for a in range(K // L):
        plsc.store_scatter(outb_v, [Ks[a]], Vv[a])


NUM_CORES = 2
NUM_SUBCORES = 16


def kernel(tens):
    nw = NUM_CORES * NUM_SUBCORES
    rows_per = R // nw
    mesh = plsc.VectorSubcoreMesh(
        core_axis_name="c", subcore_axis_name="s",
        num_cores=NUM_CORES, num_subcores=NUM_SUBCORES)

    @functools.partial(
        pl.kernel,
        mesh=mesh,
        out_type=jax.ShapeDtypeStruct((R, K), jnp.float32),
        scratch_types=[
            pltpu.VMEM((N,), jnp.float32),
            pltpu.VMEM((NCH,), jnp.float32),
            pltpu.SMEM((K,), jnp.float32),
            pltpu.SMEM((K,), jnp.int32),
            pltpu.VMEM((K,), jnp.float32),
        ],
        compiler_params=pltpu.CompilerParams(needs_layout_passes=False),
    )
    def run(tens_hbm, out_hbm, row_v, cmax_v, vals_s, poss_s, outb_v):
        wid = lax.axis_index("s") * NUM_CORES + lax.axis_index("c")

        def row_body(j, carry):
            r = wid * rows_per + j
            pltpu.sync_copy(tens_hbm.at[r], row_v)
            _row_topk(row_v, cmax_v, vals_s, poss_s, outb_v)
            pltpu.sync_copy(outb_v, out_hbm.at[r])
            return carry

        lax.fori_loop(0, rows_per, row_body, 0)

    return run(tens)
